# baseline (speedup 1.0000x reference)
import functools

import jax
import jax.numpy as jnp
import numpy as np
from jax import lax
from jax.experimental import pallas as pl
from jax.experimental.pallas import tpu as pltpu
from jax.experimental.pallas import tpu_sc as plsc

_NUM_USERS = 10000
_NUM_ITEMS = 40000
_N = 50000
_E = 800000
_D = 64
_H = 32
_EPS = 1e-7
_MIN_NORM = 1e-15
_B = 2048

_NC, _NS = 2, 16
_NP = 50048
_G = 64
_CG = 8
_GPT = 784
_EP = _NS * _GPT * _G
_RPT = _NP // _NS
_ZR = 184
_QT = (2 * _B) // _NS


_PR = 6256


def _pre_body(x_ref, o_ref):
    x = x_ref[...]
    x0c = x[:, 0:1]
    s = jnp.sum(x * x, axis=1, keepdims=True) - x0c * x0c
    x0 = jnp.sqrt(jnp.clip(1.0 + s, _EPS, None))
    th = jnp.maximum(x0, 1.0 + _EPS)
    ynorm = jnp.maximum(jnp.sqrt(s), _MIN_NORM)
    scale = jnp.log(th + jnp.sqrt((th - 1.0) * (th + 1.0))) / ynorm
    out = x * scale
    col = lax.broadcasted_iota(jnp.int32, out.shape, 1)
    out = jnp.where(col == 0, jnp.float32(0), out).astype(jnp.bfloat16)
    o_ref[0] = out[:, :_H]
    o_ref[1] = out[:, _H:]


def _preprocess(emb):
    return pl.pallas_call(
        _pre_body,
        grid=(_NP // _PR,),
        in_specs=[pl.BlockSpec((_PR, _D), lambda i: (i, 0))],
        out_specs=pl.BlockSpec((2, _PR, _H), lambda i: (0, i, 0)),
        out_shape=jax.ShapeDtypeStruct((2, _NP, _H), jnp.bfloat16),
    )(emb)


_NB = 8
_NF = 4
_LA = 6
_NE = 3
_NBLK = _GPT // _CG
_PK = plsc.PackFormat.INTERLEAVED


def _sc_body(emb2, srcx, dstg, wgt, gidx2, out, buf_a, buf_b,
             acc, src_v, dst_v, w_v, rows_v, rows_f, oacc_v,
             gsem, ssem, esem, zsem):
    c = lax.axis_index("c")
    s = lax.axis_index("s")
    tg0 = s * _GPT

    def _zero_rows(ref, n):
        def b(i, _):
            z = jnp.zeros((16,), jnp.float32)
            ref[i, pl.ds(0, 16)] = z
            ref[i, pl.ds(16, 16)] = z
            return 0
        lax.fori_loop(0, n, b, 0)

    _zero_rows(oacc_v, _QT)

    def rows_ref(b):
        return rows_v.at[pl.ds(b * _G, _G)]

    def rowsf_ref(b):
        return rows_f.at[pl.ds(b * _G, _G)]

    def issue_gather(cur, par, jj, b):
        pltpu.async_copy(
            cur.at[c].at[src_v.at[par, pl.ds(jj * _G, _G)]], rows_ref(b),
            gsem.at[b])

    def wait_gather(cur, par, jj, b):
        pltpu.make_async_copy(
            cur.at[c].at[src_v.at[par, pl.ds(jj * _G, _G)]], rows_ref(b),
            gsem.at[b]).wait()

    def issue_scatter(b4, par, j):
        pltpu.async_copy(rowsf_ref(b4), acc.at[dst_v.at[par, j]],
                         ssem.at[b4], add=True)

    def wait_scatter(b4):
        pltpu.make_async_copy(rowsf_ref(b4), acc.at[dst_v.at[0, 0]],
                              ssem.at[b4]).wait()

    def load_chunk(kb, par, sync):
        g0 = tg0 + kb * _CG
        wdst = w_v.at[pl.ds(par * (_CG * _G), _CG * _G)]
        if sync:
            pltpu.sync_copy(srcx.at[pl.ds(g0 * _G, _CG * _G)], src_v.at[par])
            pltpu.sync_copy(dstg.at[pl.ds(g0, _CG)], dst_v.at[par])
            pltpu.sync_copy(wgt.at[pl.ds(g0 * _G, _CG * _G)], wdst)
        else:
            pltpu.async_copy(srcx.at[pl.ds(g0 * _G, _CG * _G)], src_v.at[par],
                             esem.at[par])
            pltpu.async_copy(dstg.at[pl.ds(g0, _CG)], dst_v.at[par],
                             esem.at[par])
            pltpu.async_copy(wgt.at[pl.ds(g0 * _G, _CG * _G)], wdst,
                             esem.at[par])

    def wait_chunk(kb, par):
        g0 = tg0 + kb * _CG
        wdst = w_v.at[pl.ds(par * (_CG * _G), _CG * _G)]
        pltpu.make_async_copy(srcx.at[pl.ds(g0 * _G, _CG * _G)],
                              src_v.at[par], esem.at[par]).wait()
        pltpu.make_async_copy(dstg.at[pl.ds(g0, _CG)], dst_v.at[par],
                              esem.at[par]).wait()
        pltpu.make_async_copy(wgt.at[pl.ds(g0 * _G, _CG * _G)], wdst,
                              esem.at[par]).wait()

    def scale(b, b4, par, j):
        wbase = par * (_CG * _G) + j * _G
        roff = b * _G
        foff = b4 * _G

        @plsc.parallel_loop(0, _G, 1, unroll=8)
        def _(e):
            bw = plsc.load_gather(
                w_v, [jnp.full((16,), wbase + e, jnp.int32)])
            row = rows_v[roff + e, pl.ds(0, 2 * 16)]
            lo, hi = plsc.unpack(row, format=_PK)
            rows_f[foff + e, pl.ds(0, 16)] = lo * bw
            rows_f[foff + e, pl.ds(16, 16)] = hi * bw

    def do_block(cur, kb):
        par = lax.rem(kb, _NE)
        npar = lax.rem(kb + 1, _NE)
        nnpar = lax.rem(kb + 2, _NE)
        not_first = kb > 0
        has_next = kb < _NBLK - 1
        has_next2 = kb < _NBLK - 2
        for j in range(_CG):
            b = j % _NB
            b4 = j % _NF
            if j == 0:
                @pl.when(has_next2)
                def _():
                    load_chunk(kb + 2, nnpar, sync=False)
            if j == 1:
                @pl.when(has_next)
                def _():
                    wait_chunk(kb + 1, npar)
            if j < _NF:
                @pl.when(not_first)
                def _():
                    wait_scatter(b4)
            else:
                wait_scatter(b4)
            wait_gather(cur, par, j, b)
            scale(b, b4, par, j)
            issue_scatter(b4, par, j)
            tj = j + _LA
            b2 = tj % _NB
            if tj < _CG:
                issue_gather(cur, par, tj, b2)
            else:
                @pl.when(has_next)
                def _():
                    issue_gather(cur, npar, tj - _CG, b2)

    def layer(cur, nxt):
        _zero_rows(rows_f, _ZR)
        zd = [pltpu.async_copy(
                  rows_f.at[pl.ds(0, _ZR)],
                  acc.at[pl.ds(s * _RPT + z * _ZR, _ZR)], zsem)
              for z in range(_RPT // _ZR)]
        load_chunk(0, 0, sync=True)
        load_chunk(1, 1, sync=False)
        for d in zd:
            d.wait()
        for g in range(_LA):
            issue_gather(cur, 0, g, g)
        plsc.subcore_barrier()

        def blk(kb, _):
            do_block(cur, kb)
            return 0

        lax.fori_loop(0, _NBLK, blk, 0)
        for b4 in range(_NF):
            wait_scatter(b4)
        plsc.subcore_barrier()

        def pack_batch(z, _):
            r0 = s * _RPT + z * _ZR
            pltpu.sync_copy(acc.at[pl.ds(r0, _ZR)], rows_f.at[pl.ds(0, _ZR)])

            def pk(i, _):
                lo = rows_f[i, pl.ds(0, 16)]
                hi = rows_f[i, pl.ds(16, 16)]
                rows_v[i, pl.ds(0, 2 * 16)] = plsc.pack(lo, hi, format=_PK)
                return 0

            lax.fori_loop(0, _ZR, pk, 0)
            pltpu.sync_copy(rows_v.at[pl.ds(0, _ZR)],
                            nxt.at[c, pl.ds(r0, _ZR)])
            return 0

        lax.fori_loop(0, _RPT // _ZR, pack_batch, 0)
        plsc.subcore_barrier()

        pltpu.sync_copy(gidx2.at[pl.ds(s * _QT, _QT)],
                        src_v.at[0, pl.ds(0, _QT)])
        qd = [pltpu.async_copy(
                  nxt.at[c].at[src_v.at[0, pl.ds(q * _G, _G)]],
                  rows_ref(q), zsem)
              for q in range(4)]
        for d in qd:
            d.wait()

        def qadd(i, _):
            row = rows_v[i, pl.ds(0, 2 * 16)]
            lo, hi = plsc.unpack(row, format=_PK)
            oacc_v[i, pl.ds(0, 16)] = oacc_v[i, pl.ds(0, 16)] + lo
            oacc_v[i, pl.ds(16, 16)] = oacc_v[i, pl.ds(16, 16)] + hi
            return 0

        lax.fori_loop(0, _QT, qadd, 0)

    layer(emb2, buf_a)
    layer(buf_a, buf_b)
    layer(buf_b, buf_a)

    pltpu.sync_copy(oacc_v, out.at[c, pl.ds(s * _QT, _QT)])


@functools.lru_cache(maxsize=None)
def _make_sc_call(interpret=False):
    mesh = plsc.VectorSubcoreMesh(
        core_axis_name="c", subcore_axis_name="s",
        num_cores=_NC, num_subcores=_NS)
    return pl.kernel(
        _sc_body,
        out_type=(
            jax.ShapeDtypeStruct((_NC, 2 * _B, _H), jnp.float32),
            jax.ShapeDtypeStruct((_NC, _NP, _H), jnp.bfloat16),
            jax.ShapeDtypeStruct((_NC, _NP, _H), jnp.bfloat16),
        ),
        mesh=mesh,
        scratch_types=[
            pltpu.VMEM_SHARED((_NP, _H), jnp.float32),
            pltpu.VMEM((_NE, _CG * _G), jnp.int32),
            pltpu.VMEM((_NE, _CG, _G), jnp.int32),
            pltpu.VMEM((_NE * _CG * _G,), jnp.float32),
            pltpu.VMEM((_NB * _G, _H), jnp.bfloat16),
            pltpu.VMEM((_NF * _G, _H), jnp.float32),
            pltpu.VMEM((_QT, _H), jnp.float32),
            pltpu.SemaphoreType.DMA((_NB,)),
            pltpu.SemaphoreType.DMA((_NF,)),
            pltpu.SemaphoreType.DMA((_NE,)),
            pltpu.SemaphoreType.DMA,
        ],
        compiler_params=pltpu.CompilerParams(
            needs_layout_passes=False, use_tc_tiling_on_sc=False),
        interpret=interpret,
    )


_INVPERM32 = np.array(
    [(j // 2) if j % 2 == 0 else 16 + (j // 2) for j in range(_H)],
    dtype=np.int32)
_INVPERM64 = np.concatenate([_INVPERM32, _H + _INVPERM32])


def kernel(user_table, item_table, edge_weight, edge_index, user, pos):
    emb = jnp.concatenate([user_table, item_table], axis=0)
    emb2 = _preprocess(emb)

    src = edge_index[0].astype(jnp.int32)
    dst = edge_index[1].astype(jnp.int32)
    pad = _EP - _E
    pad_idx = (jnp.arange(pad, dtype=jnp.int32) * 97) % _N
    srcx = jnp.concatenate([src, pad_idx])
    dstg = jnp.concatenate([dst, pad_idx]).reshape(_EP // _G, _G)
    wgt = jnp.concatenate([edge_weight, jnp.zeros((pad,), jnp.float32)])
    gidx2 = jnp.concatenate([user.astype(jnp.int32),
                             pos.astype(jnp.int32) + _NUM_USERS])

    out, _, _ = _make_sc_call()(emb2, srcx, dstg, wgt, gidx2)
    res = jnp.concatenate([out[0], out[1]], axis=1)
    return res[:, _INVPERM64]


# speedup vs baseline: 1.0408x; 1.0408x over previous
import functools

import jax
import jax.numpy as jnp
from jax import lax
from jax.experimental import pallas as pl
from jax.experimental.pallas import tpu as pltpu
from jax.experimental.pallas import tpu_sc as plsc

_NUM_USERS = 10000
_NUM_ITEMS = 40000
_N = 50000
_E = 800000
_D = 64
_H = 32
_EPS = 1e-7
_MIN_NORM = 1e-15
_B = 2048

_NC, _NS = 2, 16
_NP = 50048
_G = 64
_CG = 8
_GPT = 784
_EP = _NS * _GPT * _G
_CPT = _GPT // _CG
_RPT = _NP // _NS
_ZR = 184
_QT = (2 * _B) // _NS


_PR = 3128


def _pre_body(x_ref, o_ref):
    x = x_ref[...]
    x0c = x[:, 0:1]
    s = jnp.sum(x * x, axis=1, keepdims=True) - x0c * x0c
    x0 = jnp.sqrt(jnp.clip(1.0 + s, _EPS, None))
    th = jnp.maximum(x0, 1.0 + _EPS)
    ynorm = jnp.maximum(jnp.sqrt(s), _MIN_NORM)
    scale = jnp.log(th + jnp.sqrt((th - 1.0) * (th + 1.0))) / ynorm
    out = x * scale
    col = lax.broadcasted_iota(jnp.int32, out.shape, 1)
    out = jnp.where(col == 0, jnp.float32(0), out)
    o_ref[0] = out[:, :_H]
    o_ref[1] = out[:, _H:]


def _preprocess(emb):
    return pl.pallas_call(
        _pre_body,
        grid=(_NP // _PR,),
        in_specs=[pl.BlockSpec((_PR, _D), lambda i: (i, 0))],
        out_specs=pl.BlockSpec((2, _PR, _H), lambda i: (0, i, 0)),
        out_shape=jax.ShapeDtypeStruct((2, _NP, _H), jnp.float32),
    )(emb)


_NB = 8
_LA = 6
_NE = 3
_NBLK = _GPT // _CG


def _sc_body(emb2, srcx, dstg, wgt, gidx2, out, buf_a, buf_b,
             acc, src_v, dst_v, w_v, rows_v, oacc_v,
             gsem, ssem, esem, zsem):
    c = lax.axis_index("c")
    s = lax.axis_index("s")
    tg0 = s * _GPT

    def _zero_rows(ref, n):
        def b(i, _):
            z = jnp.zeros((16,), jnp.float32)
            ref[i, pl.ds(0, 16)] = z
            ref[i, pl.ds(16, 16)] = z
            return 0
        lax.fori_loop(0, n, b, 0)

    _zero_rows(oacc_v, _QT)

    def rows_ref(b):
        return rows_v.at[pl.ds(b * _G, _G)]

    def issue_gather(cur, par, jj, b):
        pltpu.async_copy(
            cur.at[c].at[src_v.at[par, pl.ds(jj * _G, _G)]], rows_ref(b),
            gsem.at[b])

    def wait_gather(cur, par, jj, b):
        pltpu.make_async_copy(
            cur.at[c].at[src_v.at[par, pl.ds(jj * _G, _G)]], rows_ref(b),
            gsem.at[b]).wait()

    def issue_scatter(b, par, j):
        pltpu.async_copy(rows_ref(b), acc.at[dst_v.at[par, j]],
                         ssem.at[b], add=True)

    def wait_scatter(b):
        pltpu.make_async_copy(rows_ref(b), acc.at[dst_v.at[0, 0]],
                              ssem.at[b]).wait()

    def load_chunk(kb, par, sync):
        g0 = tg0 + kb * _CG
        wdst = w_v.at[pl.ds(par * (_CG * _G), _CG * _G)]
        if sync:
            pltpu.sync_copy(srcx.at[pl.ds(g0 * _G, _CG * _G)], src_v.at[par])
            pltpu.sync_copy(dstg.at[pl.ds(g0, _CG)], dst_v.at[par])
            pltpu.sync_copy(wgt.at[pl.ds(g0 * _G, _CG * _G)], wdst)
        else:
            pltpu.async_copy(srcx.at[pl.ds(g0 * _G, _CG * _G)], src_v.at[par],
                             esem.at[par])
            pltpu.async_copy(dstg.at[pl.ds(g0, _CG)], dst_v.at[par],
                             esem.at[par])
            pltpu.async_copy(wgt.at[pl.ds(g0 * _G, _CG * _G)], wdst,
                             esem.at[par])

    def wait_chunk(kb, par):
        g0 = tg0 + kb * _CG
        wdst = w_v.at[pl.ds(par * (_CG * _G), _CG * _G)]
        pltpu.make_async_copy(srcx.at[pl.ds(g0 * _G, _CG * _G)], src_v.at[par],
                              esem.at[par]).wait()
        pltpu.make_async_copy(dstg.at[pl.ds(g0, _CG)], dst_v.at[par],
                              esem.at[par]).wait()
        pltpu.make_async_copy(wgt.at[pl.ds(g0 * _G, _CG * _G)], wdst,
                              esem.at[par]).wait()

    def scale(b, par, j):
        wbase = par * (_CG * _G) + j * _G
        roff = b * _G

        @plsc.parallel_loop(0, _G // 16, 1, unroll=1)
        def _(k):
            wv = w_v[pl.ds(wbase + k * 16, 16)]
            for ee in range(16):
                r = roff + k * 16 + ee
                w1 = wv[ee]
                rows_v[r, pl.ds(0, 16)] = rows_v[r, pl.ds(0, 16)] * w1
                rows_v[r, pl.ds(16, 16)] = rows_v[r, pl.ds(16, 16)] * w1

    def do_block(cur, kb):
        par = lax.rem(kb, _NE)
        npar = lax.rem(kb + 1, _NE)
        nnpar = lax.rem(kb + 2, _NE)
        not_first = kb > 0
        has_next = kb < _NBLK - 1
        has_next2 = kb < _NBLK - 2
        for j in range(_CG):
            b = j % _NB
            if j == 0:
                @pl.when(has_next2)
                def _():
                    load_chunk(kb + 2, nnpar, sync=False)
            if j == 1:
                @pl.when(has_next)
                def _():
                    wait_chunk(kb + 1, npar)
            wait_gather(cur, par, j, b)
            scale(b, par, j)
            issue_scatter(b, par, j)
            tj = j + _LA
            b2 = tj % _NB
            if tj < _CG:
                if j < _NB - _LA:
                    @pl.when(not_first)
                    def _():
                        wait_scatter(b2)
                else:
                    wait_scatter(b2)
                issue_gather(cur, par, tj, b2)
            else:
                @pl.when(has_next)
                def _():
                    wait_scatter(b2)
                    issue_gather(cur, npar, tj - _CG, b2)

    def layer(cur, nxt):
        _zero_rows(rows_v, _ZR)
        zd = [pltpu.async_copy(
                  rows_v.at[pl.ds(0, _ZR)],
                  acc.at[pl.ds(s * _RPT + z * _ZR, _ZR)], zsem)
              for z in range(_RPT // _ZR)]
        load_chunk(0, 0, sync=True)
        load_chunk(1, 1, sync=False)
        for d in zd:
            d.wait()
        for g in range(_LA):
            issue_gather(cur, 0, g, g)
        plsc.subcore_barrier()

        def blk(kb, _):
            do_block(cur, kb)
            return 0

        lax.fori_loop(0, _NBLK, blk, 0)
        for b in range(_NB):
            wait_scatter(b)
        plsc.subcore_barrier()

        pltpu.sync_copy(acc.at[pl.ds(s * _RPT, _RPT)],
                        nxt.at[c, pl.ds(s * _RPT, _RPT)])
        plsc.subcore_barrier()

        pltpu.sync_copy(gidx2.at[pl.ds(s * _QT, _QT)],
                        src_v.at[0, pl.ds(0, _QT)])
        qd = [pltpu.async_copy(
                  nxt.at[c].at[src_v.at[0, pl.ds(q * _G, _G)]],
                  rows_ref(q), zsem)
              for q in range(4)]
        for d in qd:
            d.wait()

        def qadd(i, _):
            oacc_v[i, pl.ds(0, 16)] = (
                oacc_v[i, pl.ds(0, 16)] + rows_v[i, pl.ds(0, 16)])
            oacc_v[i, pl.ds(16, 16)] = (
                oacc_v[i, pl.ds(16, 16)] + rows_v[i, pl.ds(16, 16)])
            return 0

        lax.fori_loop(0, _QT, qadd, 0)

    layer(emb2, buf_a)
    layer(buf_a, buf_b)
    layer(buf_b, buf_a)

    pltpu.sync_copy(oacc_v, out.at[c, pl.ds(s * _QT, _QT)])


@functools.lru_cache(maxsize=None)
def _make_sc_call(interpret=False):
    mesh = plsc.VectorSubcoreMesh(
        core_axis_name="c", subcore_axis_name="s",
        num_cores=_NC, num_subcores=_NS)
    return pl.kernel(
        _sc_body,
        out_type=(
            jax.ShapeDtypeStruct((_NC, 2 * _B, _H), jnp.float32),
            jax.ShapeDtypeStruct((_NC, _NP, _H), jnp.float32),
            jax.ShapeDtypeStruct((_NC, _NP, _H), jnp.float32),
        ),
        mesh=mesh,
        scratch_types=[
            pltpu.VMEM_SHARED((_NP, _H), jnp.float32),
            pltpu.VMEM((_NE, _CG * _G), jnp.int32),
            pltpu.VMEM((_NE, _CG, _G), jnp.int32),
            pltpu.VMEM((_NE * _CG * _G,), jnp.float32),
            pltpu.VMEM((_NB * _G, _H), jnp.float32),
            pltpu.VMEM((_QT, _H), jnp.float32),
            pltpu.SemaphoreType.DMA((_NB,)),
            pltpu.SemaphoreType.DMA((_NB,)),
            pltpu.SemaphoreType.DMA((2,)),
            pltpu.SemaphoreType.DMA,
        ],
        compiler_params=pltpu.CompilerParams(
            needs_layout_passes=False, use_tc_tiling_on_sc=False),
        interpret=interpret,
    )


def kernel(user_table, item_table, edge_weight, edge_index, user, pos):
    emb = jnp.concatenate([user_table, item_table], axis=0)
    emb2 = _preprocess(emb)

    src = edge_index[0].astype(jnp.int32)
    dst = edge_index[1].astype(jnp.int32)
    pad = _EP - _E
    pad_idx = (jnp.arange(pad, dtype=jnp.int32) * 97) % _N
    src_p = jnp.concatenate([src, pad_idx])
    dst_p = jnp.concatenate([dst, pad_idx])
    w_p = jnp.concatenate([edge_weight,
                           jnp.zeros((pad,), jnp.float32)])
    srcx = src_p
    dstg = dst_p.reshape(_EP // _G, _G)
    gidx = jnp.concatenate([user.astype(jnp.int32),
                            pos.astype(jnp.int32) + _NUM_USERS])
    gidx2 = gidx

    out, _, _ = _make_sc_call()(emb2, srcx, dstg, w_p, gidx2)
    return jnp.concatenate([out[0], out[1]], axis=1)


# speedup vs baseline: 1.0511x; 1.0099x over previous
import functools

import jax
import jax.numpy as jnp
from jax import lax
from jax.experimental import pallas as pl
from jax.experimental.pallas import tpu as pltpu
from jax.experimental.pallas import tpu_sc as plsc

_NUM_USERS = 10000
_NUM_ITEMS = 40000
_N = 50000
_E = 800000
_D = 64
_H = 32
_EPS = 1e-7
_MIN_NORM = 1e-15
_B = 2048

_NC, _NS = 2, 16
_NP = 50048
_G = 64
_CG = 8
_GPT = 784
_EP = _NS * _GPT * _G
_CPT = _GPT // _CG
_RPT = _NP // _NS
_ZR = 184
_QT = (2 * _B) // _NS


_PR = 3128


def _pre_body(x_ref, o_ref):
    x = x_ref[...]
    x0c = x[:, 0:1]
    s = jnp.sum(x * x, axis=1, keepdims=True) - x0c * x0c
    x0 = jnp.sqrt(jnp.clip(1.0 + s, _EPS, None))
    th = jnp.maximum(x0, 1.0 + _EPS)
    ynorm = jnp.maximum(jnp.sqrt(s), _MIN_NORM)
    scale = jnp.log(th + jnp.sqrt((th - 1.0) * (th + 1.0))) / ynorm
    out = x * scale
    col = lax.broadcasted_iota(jnp.int32, out.shape, 1)
    out = jnp.where(col == 0, jnp.float32(0), out)
    o_ref[0] = out[:, :_H]
    o_ref[1] = out[:, _H:]


def _preprocess(emb):
    return pl.pallas_call(
        _pre_body,
        grid=(_NP // _PR,),
        in_specs=[pl.BlockSpec((_PR, _D), lambda i: (i, 0))],
        out_specs=pl.BlockSpec((2, _PR, _H), lambda i: (0, i, 0)),
        out_shape=jax.ShapeDtypeStruct((2, _NP, _H), jnp.float32),
    )(emb)


_NB = 8
_LA = 7
_NE = 3
_NBLK = _GPT // _CG


def _sc_body(emb2, srcx, dstg, wgt, gidx2, out, buf_a, buf_b,
             acc, src_v, dst_v, w_v, rows_v, oacc_v,
             gsem, ssem, esem, zsem):
    c = lax.axis_index("c")
    s = lax.axis_index("s")
    tg0 = s * _GPT

    def _zero_rows(ref, n):
        def b(i, _):
            z = jnp.zeros((16,), jnp.float32)
            ref[i, pl.ds(0, 16)] = z
            ref[i, pl.ds(16, 16)] = z
            return 0
        lax.fori_loop(0, n, b, 0)

    _zero_rows(oacc_v, _QT)

    def rows_ref(b):
        return rows_v.at[pl.ds(b * _G, _G)]

    def issue_gather(cur, par, jj, b):
        pltpu.async_copy(
            cur.at[c].at[src_v.at[par, pl.ds(jj * _G, _G)]], rows_ref(b),
            gsem.at[b])

    def wait_gather(cur, par, jj, b):
        pltpu.make_async_copy(
            cur.at[c].at[src_v.at[par, pl.ds(jj * _G, _G)]], rows_ref(b),
            gsem.at[b]).wait()

    def issue_scatter(b, par, j):
        pltpu.async_copy(rows_ref(b), acc.at[dst_v.at[par, j]],
                         ssem.at[b], add=True)

    def wait_scatter(b):
        pltpu.make_async_copy(rows_ref(b), acc.at[dst_v.at[0, 0]],
                              ssem.at[b]).wait()

    def load_chunk(kb, par, sync):
        g0 = tg0 + kb * _CG
        wdst = w_v.at[pl.ds(par * (_CG * _G), _CG * _G)]
        if sync:
            pltpu.sync_copy(srcx.at[pl.ds(g0 * _G, _CG * _G)], src_v.at[par])
            pltpu.sync_copy(dstg.at[pl.ds(g0, _CG)], dst_v.at[par])
            pltpu.sync_copy(wgt.at[pl.ds(g0 * _G, _CG * _G)], wdst)
        else:
            pltpu.async_copy(srcx.at[pl.ds(g0 * _G, _CG * _G)], src_v.at[par],
                             esem.at[par])
            pltpu.async_copy(dstg.at[pl.ds(g0, _CG)], dst_v.at[par],
                             esem.at[par])
            pltpu.async_copy(wgt.at[pl.ds(g0 * _G, _CG * _G)], wdst,
                             esem.at[par])

    def wait_chunk(kb, par):
        g0 = tg0 + kb * _CG
        wdst = w_v.at[pl.ds(par * (_CG * _G), _CG * _G)]
        pltpu.make_async_copy(srcx.at[pl.ds(g0 * _G, _CG * _G)], src_v.at[par],
                              esem.at[par]).wait()
        pltpu.make_async_copy(dstg.at[pl.ds(g0, _CG)], dst_v.at[par],
                              esem.at[par]).wait()
        pltpu.make_async_copy(wgt.at[pl.ds(g0 * _G, _CG * _G)], wdst,
                              esem.at[par]).wait()

    def scale(b, par, j):
        wbase = par * (_CG * _G) + j * _G
        roff = b * _G

        @plsc.parallel_loop(0, _G // 16, 1, unroll=1)
        def _(k):
            wv = w_v[pl.ds(wbase + k * 16, 16)]
            for ee in range(16):
                r = roff + k * 16 + ee
                w1 = wv[ee]
                rows_v[r, pl.ds(0, 16)] = rows_v[r, pl.ds(0, 16)] * w1
                rows_v[r, pl.ds(16, 16)] = rows_v[r, pl.ds(16, 16)] * w1

    def do_block(cur, kb):
        par = lax.rem(kb, _NE)
        npar = lax.rem(kb + 1, _NE)
        nnpar = lax.rem(kb + 2, _NE)
        not_first = kb > 0
        has_next = kb < _NBLK - 1
        has_next2 = kb < _NBLK - 2
        for j in range(_CG):
            b = j % _NB
            if j == 0:
                @pl.when(has_next2)
                def _():
                    load_chunk(kb + 2, nnpar, sync=False)
            if j == 1:
                @pl.when(has_next)
                def _():
                    wait_chunk(kb + 1, npar)
            wait_gather(cur, par, j, b)
            scale(b, par, j)
            issue_scatter(b, par, j)
            tj = j + _LA
            b2 = tj % _NB
            if tj < _CG:
                if j < _NB - _LA:
                    @pl.when(not_first)
                    def _():
                        wait_scatter(b2)
                else:
                    wait_scatter(b2)
                issue_gather(cur, par, tj, b2)
            else:
                @pl.when(has_next)
                def _():
                    wait_scatter(b2)
                    issue_gather(cur, npar, tj - _CG, b2)

    def layer(cur, nxt):
        _zero_rows(rows_v, _ZR)
        zd = [pltpu.async_copy(
                  rows_v.at[pl.ds(0, _ZR)],
                  acc.at[pl.ds(s * _RPT + z * _ZR, _ZR)], zsem)
              for z in range(_RPT // _ZR)]
        load_chunk(0, 0, sync=True)
        load_chunk(1, 1, sync=False)
        for d in zd:
            d.wait()
        for g in range(_LA):
            issue_gather(cur, 0, g, g)
        plsc.subcore_barrier()

        def blk(kb, _):
            do_block(cur, kb)
            return 0

        lax.fori_loop(0, _NBLK, blk, 0)
        for b in range(_NB):
            wait_scatter(b)
        plsc.subcore_barrier()

        pltpu.sync_copy(acc.at[pl.ds(s * _RPT, _RPT)],
                        nxt.at[c, pl.ds(s * _RPT, _RPT)])
        plsc.subcore_barrier()

        pltpu.sync_copy(gidx2.at[pl.ds(s * _QT, _QT)],
                        src_v.at[0, pl.ds(0, _QT)])
        qd = [pltpu.async_copy(
                  nxt.at[c].at[src_v.at[0, pl.ds(q * _G, _G)]],
                  rows_ref(q), zsem)
              for q in range(4)]
        for d in qd:
            d.wait()

        def qadd(i, _):
            oacc_v[i, pl.ds(0, 16)] = (
                oacc_v[i, pl.ds(0, 16)] + rows_v[i, pl.ds(0, 16)])
            oacc_v[i, pl.ds(16, 16)] = (
                oacc_v[i, pl.ds(16, 16)] + rows_v[i, pl.ds(16, 16)])
            return 0

        lax.fori_loop(0, _QT, qadd, 0)

    layer(emb2, buf_a)
    layer(buf_a, buf_b)
    layer(buf_b, buf_a)

    pltpu.sync_copy(oacc_v, out.at[c, pl.ds(s * _QT, _QT)])


@functools.lru_cache(maxsize=None)
def _make_sc_call(interpret=False):
    mesh = plsc.VectorSubcoreMesh(
        core_axis_name="c", subcore_axis_name="s",
        num_cores=_NC, num_subcores=_NS)
    return pl.kernel(
        _sc_body,
        out_type=(
            jax.ShapeDtypeStruct((_NC, 2 * _B, _H), jnp.float32),
            jax.ShapeDtypeStruct((_NC, _NP, _H), jnp.float32),
            jax.ShapeDtypeStruct((_NC, _NP, _H), jnp.float32),
        ),
        mesh=mesh,
        scratch_types=[
            pltpu.VMEM_SHARED((_NP, _H), jnp.float32),
            pltpu.VMEM((_NE, _CG * _G), jnp.int32),
            pltpu.VMEM((_NE, _CG, _G), jnp.int32),
            pltpu.VMEM((_NE * _CG * _G,), jnp.float32),
            pltpu.VMEM((_NB * _G, _H), jnp.float32),
            pltpu.VMEM((_QT, _H), jnp.float32),
            pltpu.SemaphoreType.DMA((_NB,)),
            pltpu.SemaphoreType.DMA((_NB,)),
            pltpu.SemaphoreType.DMA((2,)),
            pltpu.SemaphoreType.DMA,
        ],
        compiler_params=pltpu.CompilerParams(
            needs_layout_passes=False, use_tc_tiling_on_sc=False),
        interpret=interpret,
    )


def kernel(user_table, item_table, edge_weight, edge_index, user, pos):
    emb = jnp.concatenate([user_table, item_table], axis=0)
    emb2 = _preprocess(emb)

    src = edge_index[0].astype(jnp.int32)
    dst = edge_index[1].astype(jnp.int32)
    pad = _EP - _E
    pad_idx = (jnp.arange(pad, dtype=jnp.int32) * 97) % _N
    src_p = jnp.concatenate([src, pad_idx])
    dst_p = jnp.concatenate([dst, pad_idx])
    w_p = jnp.concatenate([edge_weight,
                           jnp.zeros((pad,), jnp.float32)])
    srcx = src_p
    dstg = dst_p.reshape(_EP // _G, _G)
    gidx = jnp.concatenate([user.astype(jnp.int32),
                            pos.astype(jnp.int32) + _NUM_USERS])
    gidx2 = gidx

    out, _, _ = _make_sc_call()(emb2, srcx, dstg, w_p, gidx2)
    return jnp.concatenate([out[0], out[1]], axis=1)
